# baseline (speedup 1.0000x reference)
import jax
import jax.numpy as jnp
from jax.experimental import pallas as pl

BATCH = 1024
SEQ = 200
DIM = 128
BLOCK_B = 32


def _add_kernel(x_ref, pe_ref, o_ref):
    o_ref[...] = x_ref[...] + pe_ref[...][None, :, :]


def kernel(padded_encoded_input_docs, pos_emb):
    x = padded_encoded_input_docs
    grid = (BATCH // BLOCK_B,)
    return pl.pallas_call(
        _add_kernel,
        grid=grid,
        in_specs=[
            pl.BlockSpec((BLOCK_B, SEQ, DIM), lambda i: (i, 0, 0)),
            pl.BlockSpec((SEQ, DIM), lambda i: (0, 0)),
        ],
        out_specs=pl.BlockSpec((BLOCK_B, SEQ, DIM), lambda i: (i, 0, 0)),
        out_shape=jax.ShapeDtypeStruct((BATCH, SEQ, DIM), x.dtype),
    )(x, pos_emb)


# speedup vs baseline: 1.0416x; 1.0416x over previous
import jax
import jax.numpy as jnp
from jax.experimental import pallas as pl

BATCH = 1024
SEQ = 200
DIM = 128
BLOCK_B = 128


def _add_kernel(x_ref, pe_ref, o_ref):
    o_ref[...] = x_ref[...] + pe_ref[...][None, :, :]


def kernel(padded_encoded_input_docs, pos_emb):
    x = padded_encoded_input_docs
    grid = (BATCH // BLOCK_B,)
    return pl.pallas_call(
        _add_kernel,
        grid=grid,
        in_specs=[
            pl.BlockSpec((BLOCK_B, SEQ, DIM), lambda i: (i, 0, 0)),
            pl.BlockSpec((SEQ, DIM), lambda i: (0, 0)),
        ],
        out_specs=pl.BlockSpec((BLOCK_B, SEQ, DIM), lambda i: (i, 0, 0)),
        out_shape=jax.ShapeDtypeStruct((BATCH, SEQ, DIM), x.dtype),
    )(x, pos_emb)
